# baseline (speedup 1.0000x reference)
import functools

import jax
import jax.numpy as jnp
import numpy as np
from jax import lax
from jax.experimental import pallas as pl
from jax.experimental.pallas import tpu as pltpu
from jax.experimental.pallas import tpu_sc as plsc

N = 10000
NPAD = 10240
H = 128
G = 50
GP = 64
CUTOFF = 10.0
MAX_NB = 32
BN = 128
BE = BN * MAX_NB
EPAD = NPAD * MAX_NB
SHIFT = float(np.log(2.0))
COEFF = -0.5 / float((CUTOFF / (G - 1)) ** 2)

def _ssp(x):
    return jax.nn.softplus(x) - SHIFT


def _knn_topk(pos):
    n = pos.shape[0]
    chunk = 1000
    n2 = jnp.sum(pos * pos, axis=1)
    cols = jnp.arange(n)

    def _chunk(args):
        c, rows = args
        d2 = jnp.sum(c * c, axis=1)[:, None] + n2[None, :] - 2.0 * (c @ pos.T)
        d2 = jnp.where(rows[:, None] == cols[None, :], jnp.inf, d2)
        d2 = jnp.maximum(d2, 0.0)
        neg, nb = jax.lax.top_k(-d2, MAX_NB)
        return nb.astype(jnp.int32), -neg

    nb, dnb = jax.lax.map(
        _chunk,
        (pos.reshape(n // chunk, chunk, 3), jnp.arange(n).reshape(n // chunk, chunk)),
    )
    return nb.reshape(n, MAX_NB), dnb.reshape(n, MAX_NB)


def _sc_gather(table, idx):
    B = idx.shape[0]
    D = table.shape[1]
    per_w = B // 32
    nchunk = per_w // 128
    mesh = plsc.VectorSubcoreMesh(core_axis_name="c", subcore_axis_name="s")

    @functools.partial(
        pl.kernel,
        mesh=mesh,
        out_type=jax.ShapeDtypeStruct((B, D), jnp.float32),
        scratch_types=[
            pltpu.VMEM((128,), jnp.int32),
            pltpu.VMEM((128, D), jnp.float32),
            pltpu.SemaphoreType.DMA,
        ],
    )
    def k(table_hbm, idx_hbm, out_hbm, idx_v, rows_v, sem):
        wid = lax.axis_index("s") * 2 + lax.axis_index("c")
        base = wid * per_w

        def body(c, carry):
            start = base + c * 128
            pltpu.sync_copy(idx_hbm.at[pl.ds(start, 128)], idx_v)
            pltpu.async_copy(table_hbm.at[idx_v], rows_v, sem).wait()
            pltpu.sync_copy(rows_v, out_hbm.at[pl.ds(start, 128)])
            return carry

        lax.fori_loop(0, nchunk, body, 0)

    return k(table, idx)


def _d2_body(pg_ref, pd_ref, o_ref):
    pg = pg_ref[...].reshape(BN, MAX_NB, H)
    diff = pg - pd_ref[...][:, None, :]
    o_ref[...] = jnp.sum(diff * diff, axis=2)


def _d2_call(posg, posp):
    return pl.pallas_call(
        _d2_body,
        grid=(NPAD // BN,),
        in_specs=[
            pl.BlockSpec((BE, H), lambda i: (i, 0)),
            pl.BlockSpec((BN, H), lambda i: (i, 0)),
        ],
        out_specs=pl.BlockSpec((BN, MAX_NB), lambda i: (i, 0)),
        out_shape=jax.ShapeDtypeStruct((NPAD, MAX_NB), jnp.float32),
    )(posg, posp)


def _xl_body(h_ref, w_ref, o_ref):
    o_ref[...] = jnp.dot(h_ref[...], w_ref[...], preferred_element_type=jnp.float32)


def _xl_call(h, w):
    return pl.pallas_call(
        _xl_body,
        grid=(NPAD // BN,),
        in_specs=[
            pl.BlockSpec((BN, H), lambda i: (i, 0)),
            pl.BlockSpec((H, H), lambda i: (0, 0)),
        ],
        out_specs=pl.BlockSpec((BN, H), lambda i: (i, 0)),
        out_shape=jax.ShapeDtypeStruct((NPAD, H), jnp.float32),
    )(h, w)


def _inter_body(h_ref, g_ref, d2_ref, d2e_ref, w1_ref, b1_ref, w2_ref, b2_ref,
                cw2_ref, cb2_ref, lw_ref, lb_ref, o_ref):
    d2 = d2_ref[...]
    d = jnp.sqrt(jnp.maximum(d2e_ref[...], 0.0))
    valid = (d2 < CUTOFF * CUTOFF).astype(jnp.float32)
    cdecay = 0.5 * (jnp.cos(d * np.pi / CUTOFF) + 1.0) * valid
    gidx = lax.broadcasted_iota(jnp.int32, (1, GP), 1)
    off = jnp.where(gidx < G, gidx.astype(jnp.float32) * (CUTOFF / (G - 1)), 1e6)
    ea = jnp.exp(COEFF * (d - off) ** 2)
    t = _ssp(jnp.dot(ea, w1_ref[...], preferred_element_type=jnp.float32)
             + b1_ref[...])
    w = (jnp.dot(t, w2_ref[...], preferred_element_type=jnp.float32)
         + b2_ref[...]) * cdecay
    msg = g_ref[...] * w
    agg = jnp.sum(msg.reshape(BN, MAX_NB, H), axis=1)
    t2 = _ssp(jnp.dot(agg, cw2_ref[...], preferred_element_type=jnp.float32)
              + cb2_ref[...])
    xo = jnp.dot(t2, lw_ref[...], preferred_element_type=jnp.float32) + lb_ref[...]
    o_ref[...] = h_ref[...] + xo


def _inter_call(h, gath, d2f, d2ef, w1p, b1, w2, b2, cw2, cb2, lw, lb):
    full = lambda r, c: pl.BlockSpec((r, c), lambda i: (0, 0))
    return pl.pallas_call(
        _inter_body,
        grid=(NPAD // BN,),
        in_specs=[
            pl.BlockSpec((BN, H), lambda i: (i, 0)),
            pl.BlockSpec((BE, H), lambda i: (i, 0)),
            pl.BlockSpec((BE, 1), lambda i: (i, 0)),
            pl.BlockSpec((BE, 1), lambda i: (i, 0)),
            full(GP, H), full(1, H), full(H, H), full(1, H),
            full(H, H), full(1, H), full(H, H), full(1, H),
        ],
        out_specs=pl.BlockSpec((BN, H), lambda i: (i, 0)),
        out_shape=jax.ShapeDtypeStruct((NPAD, H), jnp.float32),
    )(h, gath, d2f, d2ef, w1p, b1, w2, b2, cw2, cb2, lw, lb)


def _readout_body(h_ref, w_ref, b_ref, v_ref, c_ref, o_ref):
    pid = pl.program_id(0)
    p = _ssp(jnp.dot(h_ref[...], w_ref[...], preferred_element_type=jnp.float32)
             + b_ref[...])
    pb = p.astype(jnp.bfloat16).astype(jnp.float32)
    vb = v_ref[...].astype(jnp.bfloat16).astype(jnp.float32)
    s = jnp.sum(pb * vb, axis=1, keepdims=True) + c_ref[...]
    row = pid * BN + lax.broadcasted_iota(jnp.int32, (BN, 1), 0)
    s = jnp.where(row < N, s, 0.0)

    @pl.when(pid == 0)
    def _():
        o_ref[...] = jnp.zeros_like(o_ref)

    o_ref[...] += jnp.sum(s)


def _readout_call(h, f1w, f1b, f2wv, f2b):
    full = lambda r, c: pl.BlockSpec((r, c), lambda i: (0, 0))
    return pl.pallas_call(
        _readout_body,
        grid=(NPAD // BN,),
        in_specs=[
            pl.BlockSpec((BN, H), lambda i: (i, 0)),
            full(H, H // 2), full(1, H // 2), full(1, H // 2), full(1, 1),
        ],
        out_specs=pl.BlockSpec((1, 1), lambda i: (0, 0)),
        out_shape=jax.ShapeDtypeStruct((1, 1), jnp.float32),
    )(h, f1w, f1b, f2wv, f2b)


def kernel(z, pos, batch, emb, mlp_w1, mlp_b1, mlp_w2, mlp_b2, cw1, cw2, cb2,
           lw, lb, f1w, f1b, f2w, f2b):
    nbr, d2nb = _knn_topk(pos)
    nbr_p = jnp.zeros((NPAD, MAX_NB), jnp.int32).at[:N].set(nbr)
    d2_p = jnp.full((NPAD, MAX_NB), 1e9, jnp.float32).at[:N].set(d2nb)
    idx_flat = nbr_p.reshape(-1)
    d2f = d2_p.reshape(-1, 1)

    posp = jnp.zeros((NPAD, H), jnp.float32).at[:N, :3].set(pos)
    posg = _sc_gather(posp, idx_flat)
    d2ef = _d2_call(posg, posp).reshape(-1, 1)

    h = jnp.zeros((NPAD, H), jnp.float32).at[:N].set(emb[z])

    for i in range(3):
        w1p = jnp.zeros((GP, H), jnp.float32).at[:G].set(mlp_w1[i])
        xl = _xl_call(h, cw1[i])
        gath = _sc_gather(xl, idx_flat)
        h = _inter_call(
            h, gath, d2f, d2ef, w1p,
            mlp_b1[i][None, :], mlp_w2[i], mlp_b2[i][None, :],
            cw2[i], cb2[i][None, :], lw[i], lb[i][None, :],
        )

    return _readout_call(h, f1w, f1b[None, :], f2w[:, 0][None, :],
                         f2b.reshape(1, 1))


# speedup vs baseline: 1.2765x; 1.2765x over previous
import functools

import jax
import jax.numpy as jnp
import numpy as np
from jax import lax
from jax.experimental import pallas as pl
from jax.experimental.pallas import tpu as pltpu
from jax.experimental.pallas import tpu_sc as plsc

N = 10000
NPAD = 10240
H = 128
G = 50
GP = 64
CUTOFF = 10.0
MAX_NB = 32
BN = 128
BE = BN * MAX_NB
EPAD = NPAD * MAX_NB
SHIFT = float(np.log(2.0))
COEFF = -0.5 / float((CUTOFF / (G - 1)) ** 2)

def _ssp(x):
    return jax.nn.softplus(x) - SHIFT


BIG = 1e30
BRK = 64
NBLK = NPAD // 128
TOPB = 8


def _knn_body(xr_ref, yr_ref, zr_ref, xc_ref, yc_ref, zc_ref, oi_ref, od_ref):
    pid = pl.program_id(0)
    xr, yr, zr = xr_ref[...], yr_ref[...], zr_ref[...]
    xc, yc, zc = xc_ref[...], yc_ref[...], zc_ref[...]
    rnd = lambda a: a.astype(jnp.bfloat16).astype(jnp.float32)
    n2r = xr * xr + yr * yr + zr * zr
    n2c = xc * xc + yc * yc + zc * zc
    prod = (rnd(xr) * rnd(xc) + rnd(yr) * rnd(yc) + rnd(zr) * rnd(zc))
    d2 = jnp.maximum((n2r + n2c) - 2.0 * prod, 0.0)
    cio = lax.broadcasted_iota(jnp.int32, (BRK, NPAD), 1).astype(jnp.float32)
    rid = (pid * BRK
           + lax.broadcasted_iota(jnp.int32, (BRK, NPAD), 0)).astype(jnp.float32)
    keep = (d2 < CUTOFF * CUTOFF) & (cio != rid)
    d2 = jnp.where(keep, d2, BIG)

    d2r = d2.reshape(BRK, NBLK, 128)
    i128 = lax.broadcasted_iota(jnp.int32, (BRK, NBLK, 128), 2).astype(jnp.float32)
    boff = lax.broadcasted_iota(jnp.int32, (BRK, NBLK), 1).astype(jnp.float32) * 128.0
    vals, cols = [], []
    for _ in range(TOPB):
        m = jnp.min(d2r, axis=2)
        hit = d2r <= m[:, :, None]
        colb = jnp.min(jnp.where(hit, i128, BIG), axis=2)
        d2r = jnp.where(i128 == colb[:, :, None], BIG, d2r)
        vals.append(m)
        cols.append(colb + boff)
    cand_v = jnp.concatenate(vals, axis=1)
    cand_c = jnp.concatenate(cols, axis=1)

    sel_v, sel_c = [], []
    for _ in range(MAX_NB):
        m = jnp.min(cand_v, axis=1, keepdims=True)
        hit = cand_v <= m
        col = jnp.min(jnp.where(hit, cand_c, BIG), axis=1, keepdims=True)
        cand_v = jnp.where(cand_c == col, BIG, cand_v)
        sel_v.append(m)
        sel_c.append(col)
    od_ref[...] = jnp.concatenate(sel_v, axis=1)
    idx = jnp.minimum(jnp.concatenate(sel_c, axis=1), float(NPAD - 1))
    oi_ref[...] = idx.astype(jnp.int32)


def _knn_call(posk):
    xr = posk[:, 0:1]
    yr = posk[:, 1:2]
    zr = posk[:, 2:3]
    xc, yc, zc = xr.reshape(1, NPAD), yr.reshape(1, NPAD), zr.reshape(1, NPAD)
    rows = lambda: pl.BlockSpec((BRK, 1), lambda i: (i, 0))
    colsf = lambda: pl.BlockSpec((1, NPAD), lambda i: (0, 0))
    return pl.pallas_call(
        _knn_body,
        grid=(NPAD // BRK,),
        in_specs=[rows(), rows(), rows(), colsf(), colsf(), colsf()],
        out_specs=[
            pl.BlockSpec((BRK, MAX_NB), lambda i: (i, 0)),
            pl.BlockSpec((BRK, MAX_NB), lambda i: (i, 0)),
        ],
        out_shape=[
            jax.ShapeDtypeStruct((NPAD, MAX_NB), jnp.int32),
            jax.ShapeDtypeStruct((NPAD, MAX_NB), jnp.float32),
        ],
    )(xr, yr, zr, xc, yc, zc)


def _sc_gather(table, idx):
    B = idx.shape[0]
    D = table.shape[1]
    per_w = B // 32
    nchunk = per_w // 128
    mesh = plsc.VectorSubcoreMesh(core_axis_name="c", subcore_axis_name="s")

    @functools.partial(
        pl.kernel,
        mesh=mesh,
        out_type=jax.ShapeDtypeStruct((B, D), jnp.float32),
        scratch_types=[
            pltpu.VMEM((128,), jnp.int32),
            pltpu.VMEM((128, D), jnp.float32),
            pltpu.SemaphoreType.DMA,
        ],
    )
    def k(table_hbm, idx_hbm, out_hbm, idx_v, rows_v, sem):
        wid = lax.axis_index("s") * 2 + lax.axis_index("c")
        base = wid * per_w

        def body(c, carry):
            start = base + c * 128
            pltpu.sync_copy(idx_hbm.at[pl.ds(start, 128)], idx_v)
            pltpu.async_copy(table_hbm.at[idx_v], rows_v, sem).wait()
            pltpu.sync_copy(rows_v, out_hbm.at[pl.ds(start, 128)])
            return carry

        lax.fori_loop(0, nchunk, body, 0)

    return k(table, idx)


def _d2_body(pg_ref, pd_ref, o_ref):
    pg = pg_ref[...].reshape(BN, MAX_NB, H)
    diff = pg - pd_ref[...][:, None, :]
    o_ref[...] = jnp.sum(diff * diff, axis=2)


def _d2_call(posg, posp):
    return pl.pallas_call(
        _d2_body,
        grid=(NPAD // BN,),
        in_specs=[
            pl.BlockSpec((BE, H), lambda i: (i, 0)),
            pl.BlockSpec((BN, H), lambda i: (i, 0)),
        ],
        out_specs=pl.BlockSpec((BN, MAX_NB), lambda i: (i, 0)),
        out_shape=jax.ShapeDtypeStruct((NPAD, MAX_NB), jnp.float32),
    )(posg, posp)


def _xl_body(h_ref, w_ref, o_ref):
    o_ref[...] = jnp.dot(h_ref[...], w_ref[...], preferred_element_type=jnp.float32)


def _xl_call(h, w):
    return pl.pallas_call(
        _xl_body,
        grid=(NPAD // BN,),
        in_specs=[
            pl.BlockSpec((BN, H), lambda i: (i, 0)),
            pl.BlockSpec((H, H), lambda i: (0, 0)),
        ],
        out_specs=pl.BlockSpec((BN, H), lambda i: (i, 0)),
        out_shape=jax.ShapeDtypeStruct((NPAD, H), jnp.float32),
    )(h, w)


def _inter_body(h_ref, g_ref, d2_ref, d2e_ref, w1_ref, b1_ref, w2_ref, b2_ref,
                cw2_ref, cb2_ref, lw_ref, lb_ref, o_ref):
    d2 = d2_ref[...]
    d = jnp.sqrt(jnp.maximum(d2e_ref[...], 0.0))
    valid = (d2 < CUTOFF * CUTOFF).astype(jnp.float32)
    cdecay = 0.5 * (jnp.cos(d * np.pi / CUTOFF) + 1.0) * valid
    gidx = lax.broadcasted_iota(jnp.int32, (1, GP), 1)
    off = jnp.where(gidx < G, gidx.astype(jnp.float32) * (CUTOFF / (G - 1)), 1e6)
    ea = jnp.exp(COEFF * (d - off) ** 2)
    t = _ssp(jnp.dot(ea, w1_ref[...], preferred_element_type=jnp.float32)
             + b1_ref[...])
    w = (jnp.dot(t, w2_ref[...], preferred_element_type=jnp.float32)
         + b2_ref[...]) * cdecay
    msg = g_ref[...] * w
    agg = jnp.sum(msg.reshape(BN, MAX_NB, H), axis=1)
    t2 = _ssp(jnp.dot(agg, cw2_ref[...], preferred_element_type=jnp.float32)
              + cb2_ref[...])
    xo = jnp.dot(t2, lw_ref[...], preferred_element_type=jnp.float32) + lb_ref[...]
    o_ref[...] = h_ref[...] + xo


def _inter_call(h, gath, d2f, d2ef, w1p, b1, w2, b2, cw2, cb2, lw, lb):
    full = lambda r, c: pl.BlockSpec((r, c), lambda i: (0, 0))
    return pl.pallas_call(
        _inter_body,
        grid=(NPAD // BN,),
        in_specs=[
            pl.BlockSpec((BN, H), lambda i: (i, 0)),
            pl.BlockSpec((BE, H), lambda i: (i, 0)),
            pl.BlockSpec((BE, 1), lambda i: (i, 0)),
            pl.BlockSpec((BE, 1), lambda i: (i, 0)),
            full(GP, H), full(1, H), full(H, H), full(1, H),
            full(H, H), full(1, H), full(H, H), full(1, H),
        ],
        out_specs=pl.BlockSpec((BN, H), lambda i: (i, 0)),
        out_shape=jax.ShapeDtypeStruct((NPAD, H), jnp.float32),
    )(h, gath, d2f, d2ef, w1p, b1, w2, b2, cw2, cb2, lw, lb)


def _readout_body(h_ref, w_ref, b_ref, v_ref, c_ref, o_ref):
    pid = pl.program_id(0)
    p = _ssp(jnp.dot(h_ref[...], w_ref[...], preferred_element_type=jnp.float32)
             + b_ref[...])
    pb = p.astype(jnp.bfloat16).astype(jnp.float32)
    vb = v_ref[...].astype(jnp.bfloat16).astype(jnp.float32)
    s = jnp.sum(pb * vb, axis=1, keepdims=True) + c_ref[...]
    row = pid * BN + lax.broadcasted_iota(jnp.int32, (BN, 1), 0)
    s = jnp.where(row < N, s, 0.0)

    @pl.when(pid == 0)
    def _():
        o_ref[...] = jnp.zeros_like(o_ref)

    o_ref[...] += jnp.sum(s)


def _readout_call(h, f1w, f1b, f2wv, f2b):
    full = lambda r, c: pl.BlockSpec((r, c), lambda i: (0, 0))
    return pl.pallas_call(
        _readout_body,
        grid=(NPAD // BN,),
        in_specs=[
            pl.BlockSpec((BN, H), lambda i: (i, 0)),
            full(H, H // 2), full(1, H // 2), full(1, H // 2), full(1, 1),
        ],
        out_specs=pl.BlockSpec((1, 1), lambda i: (0, 0)),
        out_shape=jax.ShapeDtypeStruct((1, 1), jnp.float32),
    )(h, f1w, f1b, f2wv, f2b)


def kernel(z, pos, batch, emb, mlp_w1, mlp_b1, mlp_w2, mlp_b2, cw1, cw2, cb2,
           lw, lb, f1w, f1b, f2w, f2b):
    posk = jnp.full((NPAD, 3), 1e7, jnp.float32).at[:N].set(pos)
    nbr_p, d2sel = _knn_call(posk)
    idx_flat = nbr_p.reshape(-1)
    d2f = d2sel.reshape(-1, 1)

    posp = jnp.zeros((NPAD, H), jnp.float32).at[:N, :3].set(pos)
    posg = _sc_gather(posp, idx_flat)
    d2ef = _d2_call(posg, posp).reshape(-1, 1)

    h = jnp.zeros((NPAD, H), jnp.float32).at[:N].set(emb[z])

    for i in range(3):
        w1p = jnp.zeros((GP, H), jnp.float32).at[:G].set(mlp_w1[i])
        xl = _xl_call(h, cw1[i])
        gath = _sc_gather(xl, idx_flat)
        h = _inter_call(
            h, gath, d2f, d2ef, w1p,
            mlp_b1[i][None, :], mlp_w2[i], mlp_b2[i][None, :],
            cw2[i], cb2[i][None, :], lw[i], lb[i][None, :],
        )

    return _readout_call(h, f1w, f1b[None, :], f2w[:, 0][None, :],
                         f2b.reshape(1, 1))


# speedup vs baseline: 1.4296x; 1.1199x over previous
import functools

import jax
import jax.numpy as jnp
import numpy as np
from jax import lax
from jax.experimental import pallas as pl
from jax.experimental.pallas import tpu as pltpu
from jax.experimental.pallas import tpu_sc as plsc

N = 10000
NPAD = 10240
H = 128
G = 50
GP = 64
CUTOFF = 10.0
MAX_NB = 32
BN = 128
BE = BN * MAX_NB
EPAD = NPAD * MAX_NB
SHIFT = float(np.log(2.0))
COEFF = -0.5 / float((CUTOFF / (G - 1)) ** 2)

def _ssp(x):
    return jax.nn.softplus(x) - SHIFT


BIG = 1e30
BRK = 64
NBLK = NPAD // 128
TOPB = 8


def _knn_body(xr_ref, yr_ref, zr_ref, xc_ref, yc_ref, zc_ref,
              oi_ref, od_ref, oe_ref):
    pid = pl.program_id(0)
    xr, yr, zr = xr_ref[...], yr_ref[...], zr_ref[...]
    xc, yc, zc = xc_ref[...], yc_ref[...], zc_ref[...]
    rnd = lambda a: a.astype(jnp.bfloat16).astype(jnp.float32)
    n2r = xr * xr + yr * yr + zr * zr
    n2c = xc * xc + yc * yc + zc * zc
    prod = (rnd(xr) * rnd(xc) + rnd(yr) * rnd(yc) + rnd(zr) * rnd(zc))
    d2 = jnp.maximum((n2r + n2c) - 2.0 * prod, 0.0)
    cio = lax.broadcasted_iota(jnp.int32, (BRK, NPAD), 1).astype(jnp.float32)
    rid = (pid * BRK
           + lax.broadcasted_iota(jnp.int32, (BRK, NPAD), 0)).astype(jnp.float32)
    keep = (d2 < CUTOFF * CUTOFF) & (cio != rid)
    d2 = jnp.where(keep, d2, BIG)
    d2x = (xr - xc) ** 2 + (yr - yc) ** 2 + (zr - zc) ** 2

    d2r = d2.reshape(BRK, NBLK, 128)
    d2xr = d2x.reshape(BRK, NBLK, 128)
    i128 = lax.broadcasted_iota(jnp.int32, (BRK, NBLK, 128), 2).astype(jnp.float32)
    boff = lax.broadcasted_iota(jnp.int32, (BRK, NBLK), 1).astype(jnp.float32) * 128.0
    vals, cols, exs = [], [], []
    for _ in range(TOPB):
        m = jnp.min(d2r, axis=2)
        hit = d2r <= m[:, :, None]
        colb = jnp.min(jnp.where(hit, i128, BIG), axis=2)
        lane = i128 == colb[:, :, None]
        d2r = jnp.where(lane, BIG, d2r)
        vals.append(m)
        cols.append(colb + boff)
        exs.append(jnp.min(jnp.where(lane, d2xr, BIG), axis=2))
    cand_v = jnp.concatenate(vals, axis=1)
    cand_c = jnp.concatenate(cols, axis=1)
    cand_e = jnp.concatenate(exs, axis=1)

    sel_v, sel_c, sel_e = [], [], []
    for _ in range(MAX_NB):
        m = jnp.min(cand_v, axis=1, keepdims=True)
        hit = cand_v <= m
        col = jnp.min(jnp.where(hit, cand_c, BIG), axis=1, keepdims=True)
        slot = cand_c == col
        cand_v = jnp.where(slot, BIG, cand_v)
        sel_v.append(m)
        sel_c.append(col)
        sel_e.append(jnp.min(jnp.where(slot, cand_e, BIG), axis=1, keepdims=True))
    od_ref[...] = jnp.concatenate(sel_v, axis=1)
    oe_ref[...] = jnp.concatenate(sel_e, axis=1)
    idx = jnp.minimum(jnp.concatenate(sel_c, axis=1), float(NPAD - 1))
    oi_ref[...] = idx.astype(jnp.int32)


def _knn_call(posk):
    xr = posk[:, 0:1]
    yr = posk[:, 1:2]
    zr = posk[:, 2:3]
    xc, yc, zc = xr.reshape(1, NPAD), yr.reshape(1, NPAD), zr.reshape(1, NPAD)
    rows = lambda: pl.BlockSpec((BRK, 1), lambda i: (i, 0))
    colsf = lambda: pl.BlockSpec((1, NPAD), lambda i: (0, 0))
    return pl.pallas_call(
        _knn_body,
        grid=(NPAD // BRK,),
        in_specs=[rows(), rows(), rows(), colsf(), colsf(), colsf()],
        out_specs=[
            pl.BlockSpec((BRK, MAX_NB), lambda i: (i, 0)),
            pl.BlockSpec((BRK, MAX_NB), lambda i: (i, 0)),
            pl.BlockSpec((BRK, MAX_NB), lambda i: (i, 0)),
        ],
        out_shape=[
            jax.ShapeDtypeStruct((NPAD, MAX_NB), jnp.int32),
            jax.ShapeDtypeStruct((NPAD, MAX_NB), jnp.float32),
            jax.ShapeDtypeStruct((NPAD, MAX_NB), jnp.float32),
        ],
    )(xr, yr, zr, xc, yc, zc)


def _sc_gather(table, idx):
    B = idx.shape[0]
    D = table.shape[1]
    per_w = B // 32
    CHUNK = 80
    NBUF = 8
    GROUP = CHUNK * NBUF
    outer = per_w // GROUP
    mesh = plsc.VectorSubcoreMesh(core_axis_name="c", subcore_axis_name="s")

    @functools.partial(
        pl.kernel,
        mesh=mesh,
        out_type=jax.ShapeDtypeStruct((B, D), jnp.float32),
        scratch_types=[
            pltpu.VMEM((per_w,), jnp.int32),
            pltpu.VMEM((GROUP, D), jnp.float32),
            pltpu.SemaphoreType.DMA,
            pltpu.SemaphoreType.DMA,
        ],
    )
    def k(table_hbm, idx_hbm, out_hbm, idx_v, bufs, sem_g, sem_o):
        wid = lax.axis_index("s") * 2 + lax.axis_index("c")
        base = wid * per_w
        pltpu.sync_copy(idx_hbm.at[pl.ds(base, per_w)], idx_v)

        def body(o, carry):
            c0 = o * GROUP
            gs = [
                pltpu.async_copy(
                    table_hbm.at[idx_v.at[pl.ds(c0 + b * CHUNK, CHUNK)]],
                    bufs.at[pl.ds(b * CHUNK, CHUNK)],
                    sem_g,
                )
                for b in range(NBUF)
            ]
            for cp in gs:
                cp.wait()
            pltpu.async_copy(bufs, out_hbm.at[pl.ds(base + c0, GROUP)],
                             sem_o).wait()
            return carry

        lax.fori_loop(0, outer, body, 0)

    return k(table, idx)


def _d2_body(pg_ref, pd_ref, o_ref):
    pg = pg_ref[...].reshape(BN, MAX_NB, H)
    diff = pg - pd_ref[...][:, None, :]
    o_ref[...] = jnp.sum(diff * diff, axis=2)


def _d2_call(posg, posp):
    return pl.pallas_call(
        _d2_body,
        grid=(NPAD // BN,),
        in_specs=[
            pl.BlockSpec((BE, H), lambda i: (i, 0)),
            pl.BlockSpec((BN, H), lambda i: (i, 0)),
        ],
        out_specs=pl.BlockSpec((BN, MAX_NB), lambda i: (i, 0)),
        out_shape=jax.ShapeDtypeStruct((NPAD, MAX_NB), jnp.float32),
    )(posg, posp)


def _xl_body(h_ref, w_ref, o_ref):
    o_ref[...] = jnp.dot(h_ref[...], w_ref[...], preferred_element_type=jnp.float32)


def _xl_call(h, w):
    return pl.pallas_call(
        _xl_body,
        grid=(NPAD // BN,),
        in_specs=[
            pl.BlockSpec((BN, H), lambda i: (i, 0)),
            pl.BlockSpec((H, H), lambda i: (0, 0)),
        ],
        out_specs=pl.BlockSpec((BN, H), lambda i: (i, 0)),
        out_shape=jax.ShapeDtypeStruct((NPAD, H), jnp.float32),
    )(h, w)


def _inter_body(h_ref, g_ref, d2_ref, d2e_ref, w1_ref, b1_ref, w2_ref, b2_ref,
                cw2_ref, cb2_ref, lw_ref, lb_ref, o_ref):
    d2 = d2_ref[...]
    d = jnp.sqrt(jnp.maximum(d2e_ref[...], 0.0))
    valid = (d2 < CUTOFF * CUTOFF).astype(jnp.float32)
    cdecay = 0.5 * (jnp.cos(d * np.pi / CUTOFF) + 1.0) * valid
    gidx = lax.broadcasted_iota(jnp.int32, (1, GP), 1)
    off = jnp.where(gidx < G, gidx.astype(jnp.float32) * (CUTOFF / (G - 1)), 1e6)
    ea = jnp.exp(COEFF * (d - off) ** 2)
    t = _ssp(jnp.dot(ea, w1_ref[...], preferred_element_type=jnp.float32)
             + b1_ref[...])
    w = (jnp.dot(t, w2_ref[...], preferred_element_type=jnp.float32)
         + b2_ref[...]) * cdecay
    msg = g_ref[...] * w
    agg = jnp.sum(msg.reshape(BN, MAX_NB, H), axis=1)
    t2 = _ssp(jnp.dot(agg, cw2_ref[...], preferred_element_type=jnp.float32)
              + cb2_ref[...])
    xo = jnp.dot(t2, lw_ref[...], preferred_element_type=jnp.float32) + lb_ref[...]
    o_ref[...] = h_ref[...] + xo


def _inter_call(h, gath, d2f, d2ef, w1p, b1, w2, b2, cw2, cb2, lw, lb):
    full = lambda r, c: pl.BlockSpec((r, c), lambda i: (0, 0))
    return pl.pallas_call(
        _inter_body,
        grid=(NPAD // BN,),
        in_specs=[
            pl.BlockSpec((BN, H), lambda i: (i, 0)),
            pl.BlockSpec((BE, H), lambda i: (i, 0)),
            pl.BlockSpec((BE, 1), lambda i: (i, 0)),
            pl.BlockSpec((BE, 1), lambda i: (i, 0)),
            full(GP, H), full(1, H), full(H, H), full(1, H),
            full(H, H), full(1, H), full(H, H), full(1, H),
        ],
        out_specs=pl.BlockSpec((BN, H), lambda i: (i, 0)),
        out_shape=jax.ShapeDtypeStruct((NPAD, H), jnp.float32),
    )(h, gath, d2f, d2ef, w1p, b1, w2, b2, cw2, cb2, lw, lb)


def _readout_body(h_ref, w_ref, b_ref, v_ref, c_ref, o_ref):
    pid = pl.program_id(0)
    p = _ssp(jnp.dot(h_ref[...], w_ref[...], preferred_element_type=jnp.float32)
             + b_ref[...])
    pb = p.astype(jnp.bfloat16).astype(jnp.float32)
    vb = v_ref[...].astype(jnp.bfloat16).astype(jnp.float32)
    s = jnp.sum(pb * vb, axis=1, keepdims=True) + c_ref[...]
    row = pid * BN + lax.broadcasted_iota(jnp.int32, (BN, 1), 0)
    s = jnp.where(row < N, s, 0.0)

    @pl.when(pid == 0)
    def _():
        o_ref[...] = jnp.zeros_like(o_ref)

    o_ref[...] += jnp.sum(s)


def _readout_call(h, f1w, f1b, f2wv, f2b):
    full = lambda r, c: pl.BlockSpec((r, c), lambda i: (0, 0))
    return pl.pallas_call(
        _readout_body,
        grid=(NPAD // BN,),
        in_specs=[
            pl.BlockSpec((BN, H), lambda i: (i, 0)),
            full(H, H // 2), full(1, H // 2), full(1, H // 2), full(1, 1),
        ],
        out_specs=pl.BlockSpec((1, 1), lambda i: (0, 0)),
        out_shape=jax.ShapeDtypeStruct((1, 1), jnp.float32),
    )(h, f1w, f1b, f2wv, f2b)


def kernel(z, pos, batch, emb, mlp_w1, mlp_b1, mlp_w2, mlp_b2, cw1, cw2, cb2,
           lw, lb, f1w, f1b, f2w, f2b):
    posk = jnp.full((NPAD, 3), 1e7, jnp.float32).at[:N].set(pos)
    nbr_p, d2sel, d2ex = _knn_call(posk)
    idx_flat = nbr_p.reshape(-1)
    d2f = d2sel.reshape(-1, 1)
    d2ef = d2ex.reshape(-1, 1)

    h = jnp.zeros((NPAD, H), jnp.float32).at[:N].set(emb[z])

    for i in range(3):
        w1p = jnp.zeros((GP, H), jnp.float32).at[:G].set(mlp_w1[i])
        xl = _xl_call(h, cw1[i])
        gath = _sc_gather(xl, idx_flat)
        h = _inter_call(
            h, gath, d2f, d2ef, w1p,
            mlp_b1[i][None, :], mlp_w2[i], mlp_b2[i][None, :],
            cw2[i], cb2[i][None, :], lw[i], lb[i][None, :],
        )

    return _readout_call(h, f1w, f1b[None, :], f2w[:, 0][None, :],
                         f2b.reshape(1, 1))
